# baseline (speedup 1.0000x reference)
import functools

import jax
import jax.numpy as jnp
from jax import lax
from jax.experimental import pallas as pl
from jax.experimental.pallas import tpu as pltpu
from jax.experimental.pallas import tpu_sc as plsc

INPUT_DIM = 64
HIDDEN = 256
B = 16
S = 20
T = 20
VTILE = 2048

_NW = 32
_PAD = 512
_BPW = _PAD // _NW


def _sc_gather(table, idx_flat):
    mesh = plsc.ScalarSubcoreMesh(axis_name="c", num_cores=2)
    per_core = _PAD // 2
    chunk = 16

    @functools.partial(
        pl.kernel,
        mesh=mesh,
        out_type=jax.ShapeDtypeStruct((_PAD, INPUT_DIM), jnp.float32),
        scratch_types=[
            pltpu.SMEM((per_core,), jnp.int32),
            pltpu.SemaphoreType.DMA,
        ],
    )
    def gather_k(table_hbm, idx_hbm, out_hbm, idx_s, sem):
        base = lax.axis_index("c") * per_core
        pltpu.sync_copy(idx_hbm.at[pl.ds(base, per_core)], idx_s)
        for c0 in range(0, per_core, chunk):
            handles = []
            for i in range(c0, c0 + chunk):
                row = idx_s[i]
                handles.append(pltpu.async_copy(
                    table_hbm.at[pl.ds(row, 1)],
                    out_hbm.at[pl.ds(base + i, 1)], sem))
            for h in handles:
                h.wait()

    return gather_k(table, idx_flat)


def _matmul_t(a, b):
    return lax.dot_general(a, b, (((1,), (1,)), ((), ())),
                           preferred_element_type=jnp.float32)


def _seq2seq_body(src_ref, tgt_ref, ew0i, ew0h, eb0, ew1i, ew1h, eb1,
                  dw0i, dw0h, db0, dw1i, dw1h, db1, wout_ref, bout_ref,
                  out_ref, y_ref):
    @pl.when(pl.program_id(0) == 0)
    def _prologue():
        def layer(x_seq, wih_r, whh_r, b_r, h, c):
            whh = whh_r[...]
            xw = _matmul_t(x_seq, wih_r[...]) + b_r[...]
            outs = []
            for t in range(T):
                z = xw[t * B:(t + 1) * B] + _matmul_t(h, whh)
                zi = z[:, :HIDDEN]
                zf = z[:, HIDDEN:2 * HIDDEN]
                zg = z[:, 2 * HIDDEN:3 * HIDDEN]
                zo = z[:, 3 * HIDDEN:]
                c = jax.nn.sigmoid(zf) * c + jax.nn.sigmoid(zi) * jnp.tanh(zg)
                h = jax.nn.sigmoid(zo) * jnp.tanh(c)
                outs.append(h)
            return outs, h, c

        zeros = jnp.zeros((B, HIDDEN), jnp.float32)
        e0, h0, c0 = layer(src_ref[...], ew0i, ew0h, eb0, zeros, zeros)
        _, h1, c1 = layer(jnp.concatenate(e0, axis=0), ew1i, ew1h, eb1,
                          zeros, zeros)
        d0, _, _ = layer(tgt_ref[...], dw0i, dw0h, db0, h0, c0)
        d1, _, _ = layer(jnp.concatenate(d0, axis=0), dw1i, dw1h, db1,
                         h1, c1)
        rows = []
        for b in range(B):
            rows.append(jnp.concatenate([d1[t][b:b + 1, :] for t in range(T)],
                                        axis=0))
        y_ref[...] = jnp.concatenate(rows, axis=0)

    out_ref[...] = _matmul_t(y_ref[...], wout_ref[...]) + bout_ref[...]


def kernel(input_sequence, target_sequence, src_table, tgt_table,
           enc_W_ih_0, enc_W_hh_0, enc_b_0, enc_W_ih_1, enc_W_hh_1, enc_b_1,
           dec_W_ih_0, dec_W_hh_0, dec_b_0, dec_W_ih_1, dec_W_hh_1, dec_b_1,
           W_out, b_out):
    V = W_out.shape[0]
    n_tiles = pl.cdiv(V, VTILE)

    pad = jnp.zeros((_PAD - B * S,), jnp.int32)
    idx_src = jnp.concatenate([input_sequence.T.reshape(-1), pad])
    idx_tgt = jnp.concatenate([target_sequence.T.reshape(-1), pad])
    src_emb = _sc_gather(src_table, idx_src)[:B * S]
    tgt_emb = _sc_gather(tgt_table, idx_tgt)[:B * T]

    full = lambda shape: pl.BlockSpec(shape, lambda i: (0,) * len(shape))
    logits = pl.pallas_call(
        _seq2seq_body,
        grid=(n_tiles,),
        in_specs=[
            full((S * B, INPUT_DIM)),
            full((T * B, INPUT_DIM)),
            full((4 * HIDDEN, INPUT_DIM)),
            full((4 * HIDDEN, HIDDEN)),
            full((1, 4 * HIDDEN)),
            full((4 * HIDDEN, HIDDEN)),
            full((4 * HIDDEN, HIDDEN)),
            full((1, 4 * HIDDEN)),
            full((4 * HIDDEN, INPUT_DIM)),
            full((4 * HIDDEN, HIDDEN)),
            full((1, 4 * HIDDEN)),
            full((4 * HIDDEN, HIDDEN)),
            full((4 * HIDDEN, HIDDEN)),
            full((1, 4 * HIDDEN)),
            pl.BlockSpec((VTILE, HIDDEN), lambda i: (i, 0)),
            pl.BlockSpec((1, VTILE), lambda i: (0, i)),
        ],
        out_specs=pl.BlockSpec((T * B, VTILE), lambda i: (0, i)),
        out_shape=jax.ShapeDtypeStruct((T * B, V), jnp.float32),
        scratch_shapes=[pltpu.VMEM((T * B, HIDDEN), jnp.float32)],
    )(src_emb, tgt_emb,
      enc_W_ih_0, enc_W_hh_0, enc_b_0.reshape(1, -1),
      enc_W_ih_1, enc_W_hh_1, enc_b_1.reshape(1, -1),
      dec_W_ih_0, dec_W_hh_0, dec_b_0.reshape(1, -1),
      dec_W_ih_1, dec_W_hh_1, dec_b_1.reshape(1, -1),
      W_out, b_out.reshape(1, -1))
    return logits.reshape(B, T, V)


# speedup vs baseline: 1.0129x; 1.0129x over previous
import functools

import jax
import jax.numpy as jnp
from jax import lax
from jax.experimental import pallas as pl
from jax.experimental.pallas import tpu as pltpu
from jax.experimental.pallas import tpu_sc as plsc

INPUT_DIM = 64
HIDDEN = 256
B = 16
S = 20
T = 20
VTILE = 2048

_NTOK = B * S


def _sc_gather2(src_table, tgt_table, idx_src, idx_tgt):
    mesh = plsc.ScalarSubcoreMesh(axis_name="c", num_cores=2)
    chunk = 32

    @functools.partial(
        pl.kernel,
        mesh=mesh,
        out_type=(jax.ShapeDtypeStruct((_NTOK, INPUT_DIM), jnp.float32),
                  jax.ShapeDtypeStruct((_NTOK, INPUT_DIM), jnp.float32)),
        scratch_types=[
            pltpu.SMEM((_NTOK,), jnp.int32),
            pltpu.SemaphoreType.DMA,
        ],
        compiler_params=pltpu.CompilerParams(use_tc_tiling_on_sc=True),
    )
    def gather_k(src_hbm, tgt_hbm, isrc_hbm, itgt_hbm, out_src, out_tgt,
                 idx_s, sem):
        cid = lax.axis_index("c")

        def run(table_hbm, idx_hbm, out_hbm):
            pltpu.sync_copy(idx_hbm, idx_s)
            for c0 in range(0, _NTOK, chunk):
                handles = []
                for i in range(c0, c0 + chunk):
                    row = idx_s[i]
                    handles.append(pltpu.async_copy(
                        table_hbm.at[pl.ds(row, 1)],
                        out_hbm.at[pl.ds(i, 1)], sem))
                for h in handles:
                    h.wait()

        @pl.when(cid == 0)
        def _():
            run(src_hbm, isrc_hbm, out_src)

        @pl.when(cid == 1)
        def _():
            run(tgt_hbm, itgt_hbm, out_tgt)

    return gather_k(src_table, tgt_table, idx_src, idx_tgt)


def _matmul_t(a, b):
    return lax.dot_general(a, b, (((1,), (1,)), ((), ())),
                           preferred_element_type=jnp.float32)


def _seq2seq_body(src_ref, tgt_ref, ew0i, ew0h, eb0, ew1i, ew1h, eb1,
                  dw0i, dw0h, db0, dw1i, dw1h, db1, wout_ref, bout_ref,
                  out_ref, y_ref):
    @pl.when(pl.program_id(0) == 0)
    def _prologue():
        def layer(x_seq, wih_r, whh_r, b_r, h, c):
            whh = whh_r[...]
            xw = _matmul_t(x_seq, wih_r[...]) + b_r[...]
            outs = []
            for t in range(T):
                z = xw[t * B:(t + 1) * B] + _matmul_t(h, whh)
                zi = z[:, :HIDDEN]
                zf = z[:, HIDDEN:2 * HIDDEN]
                zg = z[:, 2 * HIDDEN:3 * HIDDEN]
                zo = z[:, 3 * HIDDEN:]
                c = jax.nn.sigmoid(zf) * c + jax.nn.sigmoid(zi) * jnp.tanh(zg)
                h = jax.nn.sigmoid(zo) * jnp.tanh(c)
                outs.append(h)
            return outs, h, c

        zeros = jnp.zeros((B, HIDDEN), jnp.float32)
        e0, h0, c0 = layer(src_ref[...], ew0i, ew0h, eb0, zeros, zeros)
        _, h1, c1 = layer(jnp.concatenate(e0, axis=0), ew1i, ew1h, eb1,
                          zeros, zeros)
        d0, _, _ = layer(tgt_ref[...], dw0i, dw0h, db0, h0, c0)
        d1, _, _ = layer(jnp.concatenate(d0, axis=0), dw1i, dw1h, db1,
                         h1, c1)
        rows = []
        for b in range(B):
            rows.append(jnp.concatenate([d1[t][b:b + 1, :] for t in range(T)],
                                        axis=0))
        y_ref[...] = jnp.concatenate(rows, axis=0)

    out_ref[...] = _matmul_t(y_ref[...], wout_ref[...]) + bout_ref[...]


def kernel(input_sequence, target_sequence, src_table, tgt_table,
           enc_W_ih_0, enc_W_hh_0, enc_b_0, enc_W_ih_1, enc_W_hh_1, enc_b_1,
           dec_W_ih_0, dec_W_hh_0, dec_b_0, dec_W_ih_1, dec_W_hh_1, dec_b_1,
           W_out, b_out):
    V = W_out.shape[0]
    n_tiles = pl.cdiv(V, VTILE)

    idx_src = input_sequence.T.reshape(-1)
    idx_tgt = target_sequence.T.reshape(-1)
    src_emb, tgt_emb = _sc_gather2(src_table, tgt_table, idx_src, idx_tgt)

    full = lambda shape: pl.BlockSpec(shape, lambda i: (0,) * len(shape))
    logits = pl.pallas_call(
        _seq2seq_body,
        grid=(n_tiles,),
        in_specs=[
            full((S * B, INPUT_DIM)),
            full((T * B, INPUT_DIM)),
            full((4 * HIDDEN, INPUT_DIM)),
            full((4 * HIDDEN, HIDDEN)),
            full((1, 4 * HIDDEN)),
            full((4 * HIDDEN, HIDDEN)),
            full((4 * HIDDEN, HIDDEN)),
            full((1, 4 * HIDDEN)),
            full((4 * HIDDEN, INPUT_DIM)),
            full((4 * HIDDEN, HIDDEN)),
            full((1, 4 * HIDDEN)),
            full((4 * HIDDEN, HIDDEN)),
            full((4 * HIDDEN, HIDDEN)),
            full((1, 4 * HIDDEN)),
            pl.BlockSpec((VTILE, HIDDEN), lambda i: (i, 0)),
            pl.BlockSpec((1, VTILE), lambda i: (0, i)),
        ],
        out_specs=pl.BlockSpec((T * B, VTILE), lambda i: (0, i)),
        out_shape=jax.ShapeDtypeStruct((T * B, V), jnp.float32),
        scratch_shapes=[pltpu.VMEM((T * B, HIDDEN), jnp.float32)],
    )(src_emb, tgt_emb,
      enc_W_ih_0, enc_W_hh_0, enc_b_0.reshape(1, -1),
      enc_W_ih_1, enc_W_hh_1, enc_b_1.reshape(1, -1),
      dec_W_ih_0, dec_W_hh_0, dec_b_0.reshape(1, -1),
      dec_W_ih_1, dec_W_hh_1, dec_b_1.reshape(1, -1),
      W_out, b_out.reshape(1, -1))
    return logits.reshape(B, T, V)
